# baseline (speedup 1.0000x reference)
import jax
import jax.numpy as jnp
from jax.experimental import pallas as pl

_BB = 16


def _body(w_ref, x_ref, o_ref):
    n = o_ref.shape[2]
    xb = x_ref[...].astype(jnp.bfloat16)
    m = jax.lax.dot_general(
        xb, w_ref[...], (((1,), (0,)), ((), ())),
        preferred_element_type=jnp.float32)
    for b in range(_BB):
        seg = jax.lax.slice(m, (n * b, 0), (n * b + n, m.shape[1]))
        o_ref[b] = seg.T


def kernel(x, idx, vals, W):
    B, N, C = x.shape
    F = W.shape[1]
    xr = x.reshape(B * N, C)
    wb = W.astype(jnp.bfloat16)
    return pl.pallas_call(
        _body,
        grid=(B // _BB,),
        in_specs=[
            pl.BlockSpec((C, F), lambda i: (0, 0)),
            pl.BlockSpec((_BB * N, C), lambda i: (i, 0)),
        ],
        out_specs=pl.BlockSpec((_BB, F, N), lambda i: (i, 0, 0)),
        out_shape=jax.ShapeDtypeStruct((B, F, N), x.dtype),
    )(wb, xr)


# speedup vs baseline: 1.0319x; 1.0319x over previous
import functools

import jax
import jax.numpy as jnp
from jax.experimental import pallas as pl
from jax.experimental.pallas import tpu as pltpu

_BB = 32
_NC = 8


def _body(w_ref, x_ref, o_ref, s_ref, *, nsteps, n):
    j = pl.program_id(1)
    xb = x_ref[...].reshape(_BB * _NC, x_ref.shape[2]).astype(jnp.bfloat16)
    m = jax.lax.dot_general(
        xb, w_ref[...], (((1,), (0,)), ((), ())),
        preferred_element_type=jnp.float32)
    s_ref[:, pl.ds(j * _NC, _NC), :] = m.reshape(_BB, _NC, m.shape[1])

    @pl.when(j == nsteps - 1)
    def _flush():
        o_ref[...] = jnp.transpose(s_ref[:, :n, :], (0, 2, 1))


def kernel(x, idx, vals, W):
    B, N, C = x.shape
    F = W.shape[1]
    wb = W.astype(jnp.bfloat16)
    nsteps = (N + _NC - 1) // _NC
    npad = nsteps * _NC
    return pl.pallas_call(
        functools.partial(_body, nsteps=nsteps, n=N),
        grid=(B // _BB, nsteps),
        in_specs=[
            pl.BlockSpec((C, F), lambda i, j: (0, 0)),
            pl.BlockSpec((_BB, _NC, C), lambda i, j: (i, j, 0)),
        ],
        out_specs=pl.BlockSpec((_BB, F, N), lambda i, j: (i, 0, 0)),
        out_shape=jax.ShapeDtypeStruct((B, F, N), x.dtype),
        scratch_shapes=[pltpu.VMEM((_BB, npad, F), jnp.float32)],
    )(wb, x)


# speedup vs baseline: 1.3173x; 1.2766x over previous
import functools

import jax
import jax.numpy as jnp
from jax.experimental import pallas as pl
from jax.experimental.pallas import tpu as pltpu

_BB = 128
_NC = 8


def _body(w_ref, x_ref, o_ref, s_ref, *, nsteps, n):
    j = pl.program_id(1)
    xb = x_ref[...].reshape(_BB * _NC, x_ref.shape[2]).astype(jnp.bfloat16)
    m = jax.lax.dot_general(
        xb, w_ref[...], (((1,), (0,)), ((), ())),
        preferred_element_type=jnp.float32)
    s_ref[:, pl.ds(j * _NC, _NC), :] = m.reshape(_BB, _NC, m.shape[1])

    @pl.when(j == nsteps - 1)
    def _flush():
        o_ref[...] = jnp.transpose(s_ref[:, :n, :], (0, 2, 1))


def kernel(x, idx, vals, W):
    B, N, C = x.shape
    F = W.shape[1]
    wb = W.astype(jnp.bfloat16)
    nsteps = (N + _NC - 1) // _NC
    npad = nsteps * _NC
    return pl.pallas_call(
        functools.partial(_body, nsteps=nsteps, n=N),
        grid=(B // _BB, nsteps),
        in_specs=[
            pl.BlockSpec((C, F), lambda i, j: (0, 0)),
            pl.BlockSpec((_BB, _NC, C), lambda i, j: (i, j, 0)),
        ],
        out_specs=pl.BlockSpec((_BB, F, N), lambda i, j: (i, 0, 0)),
        out_shape=jax.ShapeDtypeStruct((B, F, N), x.dtype),
        scratch_shapes=[pltpu.VMEM((_BB, npad, F), jnp.float32)],
    )(wb, x)


# speedup vs baseline: 1.3758x; 1.0444x over previous
import functools

import jax
import jax.numpy as jnp
from jax.experimental import pallas as pl
from jax.experimental.pallas import tpu as pltpu

_K = 8
_BB = 32
_NC = 8


def _body(w_ref, *refs, nsteps, n):
    xrefs = refs[:_K]
    o_ref = refs[_K]
    s_ref = refs[_K + 1]
    j = pl.program_id(1)
    w = w_ref[...]
    for k in range(_K):
        xb = xrefs[k][...].reshape(_BB * _NC, xrefs[k].shape[2])
        m = jax.lax.dot_general(
            xb.astype(jnp.bfloat16), w, (((1,), (0,)), ((), ())),
            preferred_element_type=jnp.float32)
        s_ref[pl.ds(k * _BB, _BB), pl.ds(j * _NC, _NC), :] = (
            m.reshape(_BB, _NC, m.shape[1]))

    @pl.when(j == nsteps - 1)
    def _flush():
        o_ref[...] = jnp.transpose(s_ref[:, :n, :], (0, 2, 1))


def kernel(x, idx, vals, W):
    B, N, C = x.shape
    F = W.shape[1]
    wb = W.astype(jnp.bfloat16)
    nsteps = (N + _NC - 1) // _NC
    npad = nsteps * _NC
    gb = B // (_K * _BB)

    def xmap(k):
        return lambda i, j: (_K * i + k, j, 0)

    return pl.pallas_call(
        functools.partial(_body, nsteps=nsteps, n=N),
        grid=(gb, nsteps),
        in_specs=[pl.BlockSpec((C, F), lambda i, j: (0, 0))] +
                 [pl.BlockSpec((_BB, _NC, C), xmap(k)) for k in range(_K)],
        out_specs=pl.BlockSpec((_K * _BB, F, N), lambda i, j: (i, 0, 0)),
        out_shape=jax.ShapeDtypeStruct((B, F, N), x.dtype),
        scratch_shapes=[pltpu.VMEM((_K * _BB, npad, F), jnp.float32)],
    )(wb, *([x] * _K))


# speedup vs baseline: 6.1132x; 4.4433x over previous
import jax
import jax.numpy as jnp
from jax.experimental import pallas as pl


def _body(wt_ref, x_ref, o_ref):
    o_ref[0] = jax.lax.dot_general(
        wt_ref[...], x_ref[0], (((1,), (0,)), ((), ())),
        preferred_element_type=jnp.float32)


def kernel(x, idx, vals, W):
    B, N, C = x.shape
    F = W.shape[1]
    xt = jnp.transpose(x, (1, 2, 0))
    wt = W.T
    out_t = pl.pallas_call(
        _body,
        grid=(N,),
        in_specs=[
            pl.BlockSpec((F, C), lambda i: (0, 0)),
            pl.BlockSpec((1, C, B), lambda i: (i, 0, 0)),
        ],
        out_specs=pl.BlockSpec((1, F, B), lambda i: (i, 0, 0)),
        out_shape=jax.ShapeDtypeStruct((N, F, B), x.dtype),
    )(wt, xt)
    return jnp.transpose(out_t, (2, 1, 0))


# speedup vs baseline: 6.8340x; 1.1179x over previous
import jax
import jax.numpy as jnp
from jax.experimental import pallas as pl

_K = 2


def _body(wt_ref, *refs):
    xrefs = refs[:_K]
    o_ref = refs[_K]
    w = wt_ref[...]
    for k in range(_K):
        o_ref[k] = jax.lax.dot_general(
            w, xrefs[k][0], (((1,), (0,)), ((), ())),
            preferred_element_type=jnp.float32)


def kernel(x, idx, vals, W):
    B, N, C = x.shape
    F = W.shape[1]
    xt = jnp.transpose(x, (1, 2, 0))
    wt = W.T

    def xmap(k):
        return lambda i: (_K * i + k, 0, 0)

    out_t = pl.pallas_call(
        _body,
        grid=(N // _K,),
        in_specs=[pl.BlockSpec((F, C), lambda i: (0, 0))] +
                 [pl.BlockSpec((1, C, B), xmap(k)) for k in range(_K)],
        out_specs=pl.BlockSpec((_K, F, B), lambda i: (i, 0, 0)),
        out_shape=jax.ShapeDtypeStruct((N, F, B), x.dtype),
    )(wt, *([xt] * _K))
    return jnp.transpose(out_t, (2, 1, 0))
